# baseline (speedup 1.0000x reference)
import functools
import numpy as np
import jax
import jax.numpy as jnp
from jax import lax
from jax.experimental import pallas as pl
from jax.experimental.pallas import tpu as pltpu
from jax.experimental.pallas import tpu_sc as plsc

N = 64
LANES = 16
NUM_WORKERS = 32
HALF = N // 2


def _build_mask():
    mask = np.zeros((N, N), dtype=bool)
    bands = [(0, 1)] + [(d, 1) for d in range(1, 16)]
    bands += [(17 + 2 * k, 2) for k in range(8)]
    bands += [(35 + 4 * k, 4) for k in range(8)]
    for d, s in bands:
        i = np.arange(0, N - d, s)
        mask[i, i + d] = True
    return mask


_MASK2D = _build_mask()
_MASKF = np.zeros(N * N + LANES, dtype=np.float32)
_MASKF[:N * N] = _MASK2D.astype(np.float32).reshape(-1)


def _sc_body(xt_hbm, maskf_hbm, outc_hbm, xtv, maskv, bufs, sems):
    ndc = xt_hbm.shape[2] // LANES
    wid = lax.axis_index("s") * 2 + lax.axis_index("c")
    b = wid // 4
    r = wid % 4

    pltpu.sync_copy(xt_hbm.at[b], xtv)
    pltpu.sync_copy(maskf_hbm, maskv)

    zero = jnp.zeros((LANES,), jnp.float32)

    def zbody(j, c):
        for buf in bufs:
            for ch in range(ndc):
                buf[j, pl.ds(ch * LANES, LANES)] = zero
        return c

    lax.fori_loop(0, HALF, zbody, 0)

    cl, ch_ = bufs

    def make_jbody(buf_c, j0, i):
        def jbody(j, R):
            m = maskv[pl.ds(i * N + j, LANES)][0]
            Rn = []
            for c in range(ndc):
                xj = xtv[j, pl.ds(c * LANES, LANES)]
                rc = jnp.maximum(R[c], xj)
                Rn.append(rc)
                buf_c[j - j0, pl.ds(c * LANES, LANES)] = rc * m
            return tuple(Rn)
        return jbody

    def slab(k, carry):
        i = r + 4 * (15 - k)

        @pl.when(k > 0)
        def _wait_low():
            pltpu.make_async_copy(cl, outc_hbm.at[b, i, pl.ds(0, HALF), :],
                                  sems.at[0]).wait()

        R0 = tuple(xtv[i, pl.ds(c * LANES, LANES)] for c in range(ndc))
        R1 = lax.fori_loop(jnp.minimum(i, HALF), HALF,
                           make_jbody(cl, 0, i), R0)
        pltpu.async_copy(cl, outc_hbm.at[b, i, pl.ds(0, HALF), :], sems.at[0])

        @pl.when(k > 0)
        def _wait_high():
            pltpu.make_async_copy(ch_, outc_hbm.at[b, i, pl.ds(HALF, HALF), :],
                                  sems.at[1]).wait()

        lax.fori_loop(jnp.maximum(i, HALF), N,
                      make_jbody(ch_, HALF, i), R1)
        pltpu.async_copy(ch_, outc_hbm.at[b, i, pl.ds(HALF, HALF), :],
                         sems.at[1])
        return carry

    lax.fori_loop(0, 16, slab, 0)

    i_last = r
    pltpu.make_async_copy(cl, outc_hbm.at[b, i_last, pl.ds(0, HALF), :],
                          sems.at[0]).wait()
    pltpu.make_async_copy(ch_, outc_hbm.at[b, i_last, pl.ds(HALF, HALF), :],
                          sems.at[1]).wait()


IBLK = 32


def _tc_boundary_body(xb_ref, xi_ref, mh_ref, out_ref):
    m = mh_ref[...][:, :, None]
    out_ref[0] = (xi_ref[0][:, None, :] + xb_ref[0][None, :, :]) * m


def kernel(x):
    B, D, n = x.shape
    xt = jnp.transpose(x, (0, 2, 1))
    maskf = jnp.asarray(_MASKF)

    sc_call = functools.partial(
        pl.kernel,
        mesh=plsc.VectorSubcoreMesh(core_axis_name="c", subcore_axis_name="s"),
        out_type=[
            jax.ShapeDtypeStruct((B, n, n, D), jnp.float32),
        ],
        scratch_types=[
            pltpu.VMEM((n, D), jnp.float32),
            pltpu.VMEM((_MASKF.shape[0],), jnp.float32),
            [pltpu.VMEM((HALF, D), jnp.float32) for _ in range(2)],
            pltpu.SemaphoreType.DMA((2,)),
        ],
        compiler_params=pltpu.CompilerParams(needs_layout_passes=False),
    )
    (outc,) = sc_call(_sc_body)(xt, maskf)

    maskh = jnp.asarray(_MASK2D.astype(np.float32) * 0.5)
    outb = pl.pallas_call(
        _tc_boundary_body,
        grid=(B, n // IBLK),
        in_specs=[
            pl.BlockSpec((1, n, D), lambda b, i: (b, 0, 0)),
            pl.BlockSpec((1, IBLK, D), lambda b, i: (b, i, 0)),
            pl.BlockSpec((IBLK, n), lambda b, i: (i, 0)),
        ],
        out_specs=pl.BlockSpec((1, IBLK, n, D), lambda b, i: (b, i, 0, 0)),
        out_shape=jax.ShapeDtypeStruct((B, n, n, D), jnp.float32),
    )(xt, xt, maskh)

    boundary = jnp.transpose(outb, (0, 3, 1, 2))
    content = jnp.transpose(outc, (0, 3, 1, 2))
    mask2d = jnp.broadcast_to(jnp.asarray(_MASK2D)[None, None], (B, 1, n, n))
    return (boundary, content, mask2d)


# speedup vs baseline: 1.0417x; 1.0417x over previous
import functools
import numpy as np
import jax
import jax.numpy as jnp
from jax import lax
from jax.experimental import pallas as pl
from jax.experimental.pallas import tpu as pltpu
from jax.experimental.pallas import tpu_sc as plsc

N = 64
LANES = 16
NUM_WORKERS = 32
HALF = N // 2


def _build_mask():
    mask = np.zeros((N, N), dtype=bool)
    bands = [(0, 1)] + [(d, 1) for d in range(1, 16)]
    bands += [(17 + 2 * k, 2) for k in range(8)]
    bands += [(35 + 4 * k, 4) for k in range(8)]
    for d, s in bands:
        i = np.arange(0, N - d, s)
        mask[i, i + d] = True
    return mask


_MASK2D = _build_mask()
_MASKF = np.zeros(N * N + LANES, dtype=np.float32)
_MASKF[:N * N] = _MASK2D.astype(np.float32).reshape(-1)


def _sc_body(xt_hbm, maskf_hbm, outc_hbm, xtv, maskv, bufs, sems):
    ndc = xt_hbm.shape[2] // LANES
    wid = lax.axis_index("s") * 2 + lax.axis_index("c")
    b = wid // 4
    r = wid % 4

    pltpu.sync_copy(xt_hbm.at[b], xtv)
    pltpu.sync_copy(maskf_hbm, maskv)

    zero = jnp.zeros((LANES,), jnp.float32)

    def zbody(j, c):
        for buf in bufs:
            for ch in range(ndc):
                buf[j, pl.ds(ch * LANES, LANES)] = zero
        return c

    lax.fori_loop(0, HALF, zbody, 0)

    cl, ch_ = bufs

    def make_jbody(buf_c, j0, i):
        def jbody(j, R):
            m = maskv[pl.ds(i * N + j, LANES)][0]
            Rn = []
            for c in range(ndc):
                xj = xtv[j, pl.ds(c * LANES, LANES)]
                rc = jnp.maximum(R[c], xj)
                Rn.append(rc)
                buf_c[j - j0, pl.ds(c * LANES, LANES)] = rc * m
            return tuple(Rn)
        return jbody

    def slab(k, carry):
        i = r + 4 * (15 - k)

        @pl.when(k > 0)
        def _wait_low():
            pltpu.make_async_copy(cl, outc_hbm.at[b, i, pl.ds(0, HALF), :],
                                  sems.at[0]).wait()

        R0 = tuple(xtv[i, pl.ds(c * LANES, LANES)] for c in range(ndc))
        R1 = lax.fori_loop(jnp.minimum(i, HALF), HALF,
                           make_jbody(cl, 0, i), R0)
        pltpu.async_copy(cl, outc_hbm.at[b, i, pl.ds(0, HALF), :], sems.at[0])

        @pl.when(k > 0)
        def _wait_high():
            pltpu.make_async_copy(ch_, outc_hbm.at[b, i, pl.ds(HALF, HALF), :],
                                  sems.at[1]).wait()

        lax.fori_loop(jnp.maximum(i, HALF), N,
                      make_jbody(ch_, HALF, i), R1)
        pltpu.async_copy(ch_, outc_hbm.at[b, i, pl.ds(HALF, HALF), :],
                         sems.at[1])
        return carry

    lax.fori_loop(0, 16, slab, 0)

    i_last = r
    pltpu.make_async_copy(cl, outc_hbm.at[b, i_last, pl.ds(0, HALF), :],
                          sems.at[0]).wait()
    pltpu.make_async_copy(ch_, outc_hbm.at[b, i_last, pl.ds(HALF, HALF), :],
                          sems.at[1]).wait()


IBLK = 16


def _tc_boundary_body(xb_ref, xi_ref, mh_ref, out_ref):
    m = mh_ref[...][:, :, None]
    out_ref[0] = (xi_ref[0][:, None, :] + xb_ref[0][None, :, :]) * m


def kernel(x):
    B, D, n = x.shape
    xt = jnp.transpose(x, (0, 2, 1))
    maskf = jnp.asarray(_MASKF)

    sc_call = functools.partial(
        pl.kernel,
        mesh=plsc.VectorSubcoreMesh(core_axis_name="c", subcore_axis_name="s"),
        out_type=[
            jax.ShapeDtypeStruct((B, n, n, D), jnp.float32),
        ],
        scratch_types=[
            pltpu.VMEM((n, D), jnp.float32),
            pltpu.VMEM((_MASKF.shape[0],), jnp.float32),
            [pltpu.VMEM((HALF, D), jnp.float32) for _ in range(2)],
            pltpu.SemaphoreType.DMA((2,)),
        ],
        compiler_params=pltpu.CompilerParams(needs_layout_passes=False),
    )
    (outc,) = sc_call(_sc_body)(xt, maskf)

    maskh = jnp.asarray(_MASK2D.astype(np.float32) * 0.5)
    outb = pl.pallas_call(
        _tc_boundary_body,
        grid=(B, n // IBLK),
        in_specs=[
            pl.BlockSpec((1, n, D), lambda b, i: (b, 0, 0)),
            pl.BlockSpec((1, IBLK, D), lambda b, i: (b, i, 0)),
            pl.BlockSpec((IBLK, n), lambda b, i: (i, 0)),
        ],
        out_specs=pl.BlockSpec((1, IBLK, n, D), lambda b, i: (b, i, 0, 0)),
        out_shape=jax.ShapeDtypeStruct((B, n, n, D), jnp.float32),
    )(xt, xt, maskh)

    boundary = jnp.transpose(outb, (0, 3, 1, 2))
    content = jnp.transpose(outc, (0, 3, 1, 2))
    mask2d = jnp.broadcast_to(jnp.asarray(_MASK2D)[None, None], (B, 1, n, n))
    return (boundary, content, mask2d)
